# Initial kernel scaffold; baseline (speedup 1.0000x reference)
import jax
import jax.numpy as jnp
from jax.experimental import pallas as pl


def kernel(x, batch, y, W1, b1, W2, b2, W3, b3):
    raise NotImplementedError("write your pallas kernel here")


# speedup vs baseline: 8.8245x; 8.8245x over previous
import functools

import jax
import jax.numpy as jnp
import numpy as np
from jax.experimental import pallas as pl
from jax.experimental.pallas import tpu as pltpu

N = 8192
K = 32
D = 128
R = 128
C = 1024
NCHUNK = N // C
IMAX = np.int32(0x7FFFFFFF)
_PREC = jax.lax.Precision.HIGHEST
_PREC_REF = jax.lax.Precision.DEFAULT


def _layer_kernel(hr_ref, ha_ref, brow_ref, bcol_ref, w_ref, b_ref, y_ref,
                  out_ref, loss_ref, keys_ref, *, relu, last):
    i = pl.program_id(0)
    hr = hr_ref[...]
    sqr = jnp.sum(hr * hr, axis=1, keepdims=True)
    br = brow_ref[...]
    ones = jnp.ones((1, D), jnp.float32)

    for ci in range(NCHUNK):
        ha_c = ha_ref[pl.ds(ci * C, C), :]
        g = jax.lax.dot_general(hr, ha_c, (((1,), (1,)), ((), ())),
                                preferred_element_type=jnp.float32,
                                precision=_PREC_REF)
        sqc = jax.lax.dot_general(ones, ha_c * ha_c, (((1,), (1,)), ((), ())),
                                  preferred_element_type=jnp.float32,
                                  precision=_PREC)
        dist = sqr + sqc - 2.0 * g
        u = jax.lax.bitcast_convert_type(dist, jnp.int32)
        key = u ^ ((u >> 31) & IMAX)
        bc = bcol_ref[0:1, pl.ds(ci * C, C)]
        col_ids = ci * C + jax.lax.broadcasted_iota(jnp.int32, (R, C), 1)
        row_ids = i * R + jax.lax.broadcasted_iota(jnp.int32, (R, C), 0)
        valid = (br == bc) & (col_ids != row_ids)
        keys_ref[:, pl.ds(ci * C, C)] = jnp.where(valid, key, IMAX)

    def count_lt(t):
        c = jnp.zeros((R, 1), jnp.int32)
        for ci in range(NCHUNK):
            kc = keys_ref[:, pl.ds(ci * C, C)]
            c = c + jnp.sum((kc < t).astype(jnp.int32), axis=1, keepdims=True)
        return c

    c0 = count_lt(jnp.zeros((R, 1), jnp.int32))
    v0 = jnp.where(c0 >= K, jnp.full((R, 1), jnp.int32(-2**31)),
                   jnp.zeros((R, 1), jnp.int32))

    def bit_body(_, carry):
        v, bit = carry
        t = v + bit
        c = count_lt(t)
        return jnp.where(c >= K, v, t), bit >> 1

    v, _ = jax.lax.fori_loop(0, 31, bit_body, (v0, jnp.int32(2**30)))

    acc = jnp.zeros((R, D), jnp.float32)
    cnt = jnp.zeros((R, 1), jnp.float32)
    for ci in range(NCHUNK):
        kc = keys_ref[:, pl.ds(ci * C, C)]
        mc = ((kc <= v) & (kc != IMAX)).astype(jnp.float32)
        cnt = cnt + jnp.sum(mc, axis=1, keepdims=True)
        ha_c = ha_ref[pl.ds(ci * C, C), :]
        acc = acc + jax.lax.dot_general(mc, ha_c, (((1,), (0,)), ((), ())),
                                        preferred_element_type=jnp.float32,
                                        precision=_PREC)
    agg = acc / jnp.maximum(cnt, 1.0)

    out = jax.lax.dot_general(agg, w_ref[...], (((1,), (0,)), ((), ())),
                              preferred_element_type=jnp.float32,
                              precision=_PREC_REF) + b_ref[...]
    if relu:
        out = jnp.maximum(out, 0.0)
    out_ref[...] = out

    if last:
        yb = y_ref[...]
        d2 = (out - yb) ** 2
        part = jnp.sum(jnp.sum(d2, axis=1, keepdims=True), axis=0,
                       keepdims=True)
        prev = jnp.where(i == 0, jnp.zeros((1, 1), jnp.float32),
                         loss_ref[...])
        total = prev + part
        loss_ref[...] = jnp.where(i == pl.num_programs(0) - 1,
                                  total / jnp.float32(N * 3), total)


def _layer(h, brow, bcol, w, b, y, relu, last):
    kern = functools.partial(_layer_kernel, relu=relu, last=last)
    grid = (N // R,)
    in_specs = [
        pl.BlockSpec((R, D), lambda i: (i, 0)),
        pl.BlockSpec((N, D), lambda i: (0, 0)),
        pl.BlockSpec((R, 1), lambda i: (i, 0)),
        pl.BlockSpec((1, N), lambda i: (0, 0)),
        pl.BlockSpec((D, D), lambda i: (0, 0)),
        pl.BlockSpec((1, D), lambda i: (0, 0)),
        pl.BlockSpec((R, D), lambda i: (i, 0)),
    ]
    out_specs = [
        pl.BlockSpec((R, D), lambda i: (i, 0)),
        pl.BlockSpec((1, 1), lambda i: (0, 0)),
    ]
    out_shape = [
        jax.ShapeDtypeStruct((N, D), jnp.float32),
        jax.ShapeDtypeStruct((1, 1), jnp.float32),
    ]
    return pl.pallas_call(
        kern, grid=grid, in_specs=in_specs, out_specs=out_specs,
        out_shape=out_shape,
        scratch_shapes=[pltpu.VMEM((R, N), jnp.int32)],
    )(h, h, brow, bcol, w, b, y)


def _pad_w(w):
    return jnp.pad(w, ((0, D - w.shape[0]), (0, D - w.shape[1])))


def _pad_b(b):
    return jnp.pad(b, (0, D - b.shape[0])).reshape(1, D)


def kernel(x, batch, y, W1, b1, W2, b2, W3, b3):
    h = jnp.pad(x, ((0, 0), (0, D - x.shape[1])))
    yp = jnp.pad(y, ((0, 0), (0, D - y.shape[1])))
    brow = batch.reshape(N, 1)
    bcol = batch.reshape(1, N)
    h1, _ = _layer(h, brow, bcol, _pad_w(W1), _pad_b(b1), yp, True, False)
    h2, _ = _layer(h1, brow, bcol, _pad_w(W2), _pad_b(b2), yp, True, False)
    h3, loss = _layer(h2, brow, bcol, _pad_w(W3), _pad_b(b3), yp, False, True)
    return h3[:, :3], loss[0, 0]


# speedup vs baseline: 18.6760x; 2.1164x over previous
import functools

import jax
import jax.numpy as jnp
import numpy as np
from jax.experimental import pallas as pl
from jax.experimental.pallas import tpu as pltpu

N = 8192
K = 32
D = 128
R = 128
C = 1024
NCHUNK = N // C
WCHUNK = 3
ALIGN = 512
IMAX = np.int32(0x7FFFFFFF)
_PREC = jax.lax.Precision.HIGHEST
_PREC_REF = jax.lax.Precision.DEFAULT


def _phases(i, hr, sqr, br, ha_ref, bcol_ref, keys_ref, w_ref, b_ref, y_ref,
            out_ref, loss_ref, lo, nchunk, relu, last):
    ones = jnp.ones((1, D), jnp.float32)

    for ci in range(nchunk):
        off = pl.multiple_of(lo + ci * C, ALIGN)
        ha_c = ha_ref[pl.ds(off, C), :]
        g = jax.lax.dot_general(hr, ha_c, (((1,), (1,)), ((), ())),
                                preferred_element_type=jnp.float32,
                                precision=_PREC_REF)
        sqc = jax.lax.dot_general(ones, ha_c * ha_c, (((1,), (1,)), ((), ())),
                                  preferred_element_type=jnp.float32,
                                  precision=_PREC)
        dist = sqr + sqc - 2.0 * g
        u = jax.lax.bitcast_convert_type(dist, jnp.int32)
        key = u ^ ((u >> 31) & IMAX)
        bc = bcol_ref[0:1, pl.ds(off, C)]
        col_ids = off + jax.lax.broadcasted_iota(jnp.int32, (R, C), 1)
        row_ids = i * R + jax.lax.broadcasted_iota(jnp.int32, (R, C), 0)
        valid = (br == bc) & (col_ids != row_ids)
        keys_ref[:, ci * C:(ci + 1) * C] = jnp.where(valid, key, IMAX)

    def count_lt(t):
        c = jnp.zeros((R, 1), jnp.int32)
        for ci in range(nchunk):
            kc = keys_ref[:, ci * C:(ci + 1) * C]
            c = c + jnp.sum((kc < t).astype(jnp.int32), axis=1, keepdims=True)
        return c

    c0 = count_lt(jnp.zeros((R, 1), jnp.int32))
    v0 = jnp.where(c0 >= K, jnp.full((R, 1), jnp.int32(-2**31)),
                   jnp.zeros((R, 1), jnp.int32))

    def bit_body(_, carry):
        v, bit = carry
        t = v + bit
        c = count_lt(t)
        return jnp.where(c >= K, v, t), bit >> 1

    v, _ = jax.lax.fori_loop(0, 31, bit_body, (v0, jnp.int32(2**30)))

    acc = jnp.zeros((R, D), jnp.float32)
    cnt = jnp.zeros((R, 1), jnp.float32)
    for ci in range(nchunk):
        kc = keys_ref[:, ci * C:(ci + 1) * C]
        mc = ((kc <= v) & (kc != IMAX)).astype(jnp.float32)
        cnt = cnt + jnp.sum(mc, axis=1, keepdims=True)
        ha_c = ha_ref[pl.ds(pl.multiple_of(lo + ci * C, ALIGN), C), :]
        acc = acc + jax.lax.dot_general(mc, ha_c, (((1,), (0,)), ((), ())),
                                        preferred_element_type=jnp.float32,
                                        precision=_PREC)
    agg = acc / jnp.maximum(cnt, 1.0)

    out = jax.lax.dot_general(agg, w_ref[...], (((1,), (0,)), ((), ())),
                              preferred_element_type=jnp.float32,
                              precision=_PREC_REF) + b_ref[...]
    if relu:
        out = jnp.maximum(out, 0.0)
    out_ref[...] = out

    if last:
        yb = y_ref[...]
        d2 = (out - yb) ** 2
        part = jnp.sum(jnp.sum(d2, axis=1, keepdims=True), axis=0,
                       keepdims=True)
        prev = jnp.where(i == 0, jnp.zeros((1, 1), jnp.float32),
                         loss_ref[...])
        total = prev + part
        loss_ref[...] = jnp.where(i == pl.num_programs(0) - 1,
                                  total / jnp.float32(N * 3), total)


def _layer_kernel(lo_ref, span_ref, hr_ref, ha_ref, brow_ref, bcol_ref,
                  w_ref, b_ref, y_ref, out_ref, loss_ref, keys_ref, *,
                  relu, last):
    i = pl.program_id(0)
    hr = hr_ref[...]
    sqr = jnp.sum(hr * hr, axis=1, keepdims=True)
    br = brow_ref[...]
    body = functools.partial(_phases, i, hr, sqr, br, ha_ref, bcol_ref,
                             keys_ref, w_ref, b_ref, y_ref, out_ref,
                             loss_ref, relu=relu, last=last)
    fits = span_ref[i] <= WCHUNK * C

    @pl.when(fits)
    def _windowed():
        body(lo=lo_ref[i], nchunk=WCHUNK)

    @pl.when(jnp.logical_not(fits))
    def _full():
        body(lo=jnp.int32(0), nchunk=NCHUNK)


def _layer(h, brow, bcol, lo_al, span_al, w, b, y, relu, last):
    kern = functools.partial(_layer_kernel, relu=relu, last=last)
    grid_spec = pltpu.PrefetchScalarGridSpec(
        num_scalar_prefetch=2,
        grid=(N // R,),
        in_specs=[
            pl.BlockSpec((R, D), lambda i, *_: (i, 0)),
            pl.BlockSpec((N, D), lambda i, *_: (0, 0)),
            pl.BlockSpec((R, 1), lambda i, *_: (i, 0)),
            pl.BlockSpec((1, N), lambda i, *_: (0, 0)),
            pl.BlockSpec((D, D), lambda i, *_: (0, 0)),
            pl.BlockSpec((1, D), lambda i, *_: (0, 0)),
            pl.BlockSpec((R, D), lambda i, *_: (i, 0)),
        ],
        out_specs=[
            pl.BlockSpec((R, D), lambda i, *_: (i, 0)),
            pl.BlockSpec((1, 1), lambda i, *_: (0, 0)),
        ],
        scratch_shapes=[pltpu.VMEM((R, N), jnp.int32)],
    )
    out_shape = [
        jax.ShapeDtypeStruct((N, D), jnp.float32),
        jax.ShapeDtypeStruct((1, 1), jnp.float32),
    ]
    return pl.pallas_call(kern, grid_spec=grid_spec, out_shape=out_shape)(
        lo_al, span_al, h, h, brow, bcol, w, b, y)


def _pad_w(w):
    return jnp.pad(w, ((0, D - w.shape[0]), (0, D - w.shape[1])))


def _pad_b(b):
    return jnp.pad(b, (0, D - b.shape[0])).reshape(1, D)


def kernel(x, batch, y, W1, b1, W2, b2, W3, b3):
    h = jnp.pad(x, ((0, 0), (0, D - x.shape[1])))
    yp = jnp.pad(y, ((0, 0), (0, D - y.shape[1])))
    brow = batch.reshape(N, 1)
    bcol = batch.reshape(1, N)
    r0 = jnp.arange(0, N, R)
    b0 = batch[r0]
    b1_ = batch[r0 + R - 1]
    lo = jnp.searchsorted(batch, b0, side="left").astype(jnp.int32)
    hi = jnp.searchsorted(batch, b1_, side="right").astype(jnp.int32)
    lo_al = (lo // ALIGN) * ALIGN
    lo_al = jnp.minimum(lo_al, N - WCHUNK * C)
    span_al = hi - lo_al
    h1, _ = _layer(h, brow, bcol, lo_al, span_al, _pad_w(W1), _pad_b(b1), yp,
                   True, False)
    h2, _ = _layer(h1, brow, bcol, lo_al, span_al, _pad_w(W2), _pad_b(b2), yp,
                   True, False)
    h3, loss = _layer(h2, brow, bcol, lo_al, span_al, _pad_w(W3), _pad_b(b3),
                      yp, False, True)
    return h3[:, :3], loss[0, 0]


# speedup vs baseline: 23.2518x; 1.2450x over previous
import functools

import jax
import jax.numpy as jnp
import numpy as np
from jax.experimental import pallas as pl
from jax.experimental.pallas import tpu as pltpu

N = 8192
K = 32
D = 128
R = 128
C = 1024
NCHUNK = N // C
WCHUNK = 3
ALIGN = 512
IMAX = np.int32(0x7FFFFFFF)
_PREC = jax.lax.Precision.HIGHEST
_PREC_REF = jax.lax.Precision.DEFAULT


def _phases(i, hr, sqr, br, ha_ref, bcol_ref, keys_ref, w_ref, b_ref, y_ref,
            out_ref, loss_ref, lo, nchunk, relu, last):
    ones = jnp.ones((1, D), jnp.float32)

    for ci in range(nchunk):
        off = pl.multiple_of(lo + ci * C, ALIGN)
        ha_c = ha_ref[pl.ds(off, C), :]
        g = jax.lax.dot_general(hr, ha_c, (((1,), (1,)), ((), ())),
                                preferred_element_type=jnp.float32,
                                precision=_PREC_REF)
        sqc = jax.lax.dot_general(ones, ha_c * ha_c, (((1,), (1,)), ((), ())),
                                  preferred_element_type=jnp.float32,
                                  precision=_PREC)
        dist = sqr + sqc - 2.0 * g
        u = jax.lax.bitcast_convert_type(dist, jnp.int32)
        key = u ^ ((u >> 31) & IMAX)
        bc = bcol_ref[0:1, pl.ds(off, C)]
        col_ids = off + jax.lax.broadcasted_iota(jnp.int32, (R, C), 1)
        row_ids = i * R + jax.lax.broadcasted_iota(jnp.int32, (R, C), 0)
        valid = (br == bc) & (col_ids != row_ids)
        keys_ref[:, ci * C:(ci + 1) * C] = jnp.where(valid, key, IMAX)

    def count_le(t):
        c = jnp.zeros((R, 1), jnp.int32)
        for ci in range(nchunk):
            kc = keys_ref[:, ci * C:(ci + 1) * C]
            c = c + jnp.sum((kc <= t).astype(jnp.int32), axis=1,
                            keepdims=True)
        return c

    mc = keys_ref[:, 0:C]
    for ci in range(1, nchunk):
        mc = jnp.minimum(mc, keys_ref[:, ci * C:(ci + 1) * C])
    w = C
    while w > 64:
        w //= 2
        mc = jnp.minimum(mc[:, :w], mc[:, w:2 * w])
    ub = jnp.max(mc, axis=1, keepdims=True)
    lb = jnp.min(mc, axis=1, keepdims=True)

    def bi_cond(carry):
        it, _, _, _, res = carry
        return jnp.logical_and(it < 34, jnp.sum(res) < R)

    def bi_body(carry):
        it, lo_, hi_, v_, res = carry
        d = hi_ - lo_
        mid = lo_ + ((d >> 1) & IMAX)
        c = count_le(mid)
        hit = jnp.logical_and(c == K, res == 0)
        v_ = jnp.where(hit, mid, v_)
        res = jnp.where(hit, jnp.int32(1), res)
        lt = c < K
        lo_ = jnp.where(lt, mid, lo_)
        hi_ = jnp.where(lt, hi_, mid)
        return it + 1, lo_, hi_, v_, res

    zero = jnp.zeros((R, 1), jnp.int32)
    _, _, hi_f, v, res_f = jax.lax.while_loop(
        bi_cond, bi_body, (jnp.int32(0), lb - 1, ub, zero, zero))
    v = jnp.where(res_f == 1, v, hi_f)

    acc = jnp.zeros((R, D), jnp.float32)
    cnt = jnp.zeros((R, 1), jnp.float32)
    for ci in range(nchunk):
        kc = keys_ref[:, ci * C:(ci + 1) * C]
        mc = ((kc <= v) & (kc != IMAX)).astype(jnp.float32)
        cnt = cnt + jnp.sum(mc, axis=1, keepdims=True)
        ha_c = ha_ref[pl.ds(pl.multiple_of(lo + ci * C, ALIGN), C), :]
        acc = acc + jax.lax.dot_general(mc, ha_c, (((1,), (0,)), ((), ())),
                                        preferred_element_type=jnp.float32,
                                        precision=_PREC)
    agg = acc / jnp.maximum(cnt, 1.0)

    out = jax.lax.dot_general(agg, w_ref[...], (((1,), (0,)), ((), ())),
                              preferred_element_type=jnp.float32,
                              precision=_PREC_REF) + b_ref[...]
    if relu:
        out = jnp.maximum(out, 0.0)
    out_ref[...] = out

    if last:
        yb = y_ref[...]
        d2 = (out - yb) ** 2
        part = jnp.sum(jnp.sum(d2, axis=1, keepdims=True), axis=0,
                       keepdims=True)
        prev = jnp.where(i == 0, jnp.zeros((1, 1), jnp.float32),
                         loss_ref[...])
        total = prev + part
        loss_ref[...] = jnp.where(i == pl.num_programs(0) - 1,
                                  total / jnp.float32(N * 3), total)


def _layer_kernel(lo_ref, span_ref, hr_ref, ha_ref, brow_ref, bcol_ref,
                  w_ref, b_ref, y_ref, out_ref, loss_ref, keys_ref, *,
                  relu, last):
    i = pl.program_id(0)
    hr = hr_ref[...]
    sqr = jnp.sum(hr * hr, axis=1, keepdims=True)
    br = brow_ref[...]
    body = functools.partial(_phases, i, hr, sqr, br, ha_ref, bcol_ref,
                             keys_ref, w_ref, b_ref, y_ref, out_ref,
                             loss_ref, relu=relu, last=last)
    fits = span_ref[i] <= WCHUNK * C

    @pl.when(fits)
    def _windowed():
        body(lo=lo_ref[i], nchunk=WCHUNK)

    @pl.when(jnp.logical_not(fits))
    def _full():
        body(lo=jnp.int32(0), nchunk=NCHUNK)


def _layer(h, brow, bcol, lo_al, span_al, w, b, y, relu, last):
    kern = functools.partial(_layer_kernel, relu=relu, last=last)
    grid_spec = pltpu.PrefetchScalarGridSpec(
        num_scalar_prefetch=2,
        grid=(N // R,),
        in_specs=[
            pl.BlockSpec((R, D), lambda i, *_: (i, 0)),
            pl.BlockSpec((N, D), lambda i, *_: (0, 0)),
            pl.BlockSpec((R, 1), lambda i, *_: (i, 0)),
            pl.BlockSpec((1, N), lambda i, *_: (0, 0)),
            pl.BlockSpec((D, D), lambda i, *_: (0, 0)),
            pl.BlockSpec((1, D), lambda i, *_: (0, 0)),
            pl.BlockSpec((R, D), lambda i, *_: (i, 0)),
        ],
        out_specs=[
            pl.BlockSpec((R, D), lambda i, *_: (i, 0)),
            pl.BlockSpec((1, 1), lambda i, *_: (0, 0)),
        ],
        scratch_shapes=[pltpu.VMEM((R, N), jnp.int32)],
    )
    out_shape = [
        jax.ShapeDtypeStruct((N, D), jnp.float32),
        jax.ShapeDtypeStruct((1, 1), jnp.float32),
    ]
    return pl.pallas_call(kern, grid_spec=grid_spec, out_shape=out_shape)(
        lo_al, span_al, h, h, brow, bcol, w, b, y)


def _pad_w(w):
    return jnp.pad(w, ((0, D - w.shape[0]), (0, D - w.shape[1])))


def _pad_b(b):
    return jnp.pad(b, (0, D - b.shape[0])).reshape(1, D)


def kernel(x, batch, y, W1, b1, W2, b2, W3, b3):
    h = jnp.pad(x, ((0, 0), (0, D - x.shape[1])))
    yp = jnp.pad(y, ((0, 0), (0, D - y.shape[1])))
    brow = batch.reshape(N, 1)
    bcol = batch.reshape(1, N)
    r0 = jnp.arange(0, N, R)
    b0 = batch[r0]
    b1_ = batch[r0 + R - 1]
    lo = jnp.searchsorted(batch, b0, side="left").astype(jnp.int32)
    hi = jnp.searchsorted(batch, b1_, side="right").astype(jnp.int32)
    lo_al = (lo // ALIGN) * ALIGN
    lo_al = jnp.minimum(lo_al, N - WCHUNK * C)
    span_al = hi - lo_al
    h1, _ = _layer(h, brow, bcol, lo_al, span_al, _pad_w(W1), _pad_b(b1), yp,
                   True, False)
    h2, _ = _layer(h1, brow, bcol, lo_al, span_al, _pad_w(W2), _pad_b(b2), yp,
                   True, False)
    h3, loss = _layer(h2, brow, bcol, lo_al, span_al, _pad_w(W3), _pad_b(b3),
                      yp, False, True)
    return h3[:, :3], loss[0, 0]


# speedup vs baseline: 25.6082x; 1.1013x over previous
import functools

import jax
import jax.numpy as jnp
import numpy as np
from jax.experimental import pallas as pl
from jax.experimental.pallas import tpu as pltpu

N = 8192
K = 32
D = 128
R = 256
C = 1024
NCHUNK = N // C
WCHUNK = 3
ALIGN = 512
IMAX = np.int32(0x7FFFFFFF)
_PREC = jax.lax.Precision.HIGHEST
_PREC_REF = jax.lax.Precision.DEFAULT


def _phases(i, hr, sqr, br, ha_ref, bcol_ref, keys_ref, w_ref, b_ref, y_ref,
            out_ref, loss_ref, lo, nchunk, relu, last):
    ones = jnp.ones((1, D), jnp.float32)

    for ci in range(nchunk):
        off = pl.multiple_of(lo + ci * C, ALIGN)
        ha_c = ha_ref[pl.ds(off, C), :]
        g = jax.lax.dot_general(hr, ha_c, (((1,), (1,)), ((), ())),
                                preferred_element_type=jnp.float32,
                                precision=_PREC_REF)
        sqc = jax.lax.dot_general(ones, ha_c * ha_c, (((1,), (1,)), ((), ())),
                                  preferred_element_type=jnp.float32,
                                  precision=_PREC)
        dist = sqr + sqc - 2.0 * g
        u = jax.lax.bitcast_convert_type(dist, jnp.int32)
        key = u ^ ((u >> 31) & IMAX)
        bc = bcol_ref[0:1, pl.ds(off, C)]
        col_ids = off + jax.lax.broadcasted_iota(jnp.int32, (R, C), 1)
        row_ids = i * R + jax.lax.broadcasted_iota(jnp.int32, (R, C), 0)
        valid = (br == bc) & (col_ids != row_ids)
        keys_ref[:, ci * C:(ci + 1) * C] = jnp.where(valid, key, IMAX)

    def count_le(t):
        c = jnp.zeros((R, 1), jnp.int32)
        for ci in range(nchunk):
            kc = keys_ref[:, ci * C:(ci + 1) * C]
            c = c + jnp.sum((kc <= t).astype(jnp.int32), axis=1,
                            keepdims=True)
        return c

    mc = keys_ref[:, 0:C]
    for ci in range(1, nchunk):
        mc = jnp.minimum(mc, keys_ref[:, ci * C:(ci + 1) * C])
    w = C
    while w > 32:
        w //= 2
        mc = jnp.minimum(mc[:, :w], mc[:, w:2 * w])
    ub = jnp.max(mc, axis=1, keepdims=True)
    lb = jnp.min(mc, axis=1, keepdims=True)

    def bi_cond(carry):
        it, _, _, _, res = carry
        return jnp.logical_and(it < 34, jnp.sum(res) < R)

    def bi_body(carry):
        it, lo_, hi_, v_, res = carry
        d = hi_ - lo_
        mid = lo_ + ((d >> 1) & IMAX)
        c = count_le(mid)
        hit = jnp.logical_and(c == K, res == 0)
        v_ = jnp.where(hit, mid, v_)
        res = jnp.where(hit, jnp.int32(1), res)
        lt = c < K
        lo_ = jnp.where(lt, mid, lo_)
        hi_ = jnp.where(lt, hi_, mid)
        return it + 1, lo_, hi_, v_, res

    zero = jnp.zeros((R, 1), jnp.int32)
    _, _, hi_f, v, res_f = jax.lax.while_loop(
        bi_cond, bi_body, (jnp.int32(0), lb - 1, ub, zero, zero))
    v = jnp.where(res_f == 1, v, hi_f)

    acc = jnp.zeros((R, D), jnp.float32)
    cnt = jnp.zeros((R, 1), jnp.float32)
    for ci in range(nchunk):
        kc = keys_ref[:, ci * C:(ci + 1) * C]
        mc = ((kc <= v) & (kc != IMAX)).astype(jnp.float32)
        cnt = cnt + jnp.sum(mc, axis=1, keepdims=True)
        ha_c = ha_ref[pl.ds(pl.multiple_of(lo + ci * C, ALIGN), C), :]
        acc = acc + jax.lax.dot_general(mc, ha_c, (((1,), (0,)), ((), ())),
                                        preferred_element_type=jnp.float32,
                                        precision=_PREC)
    agg = acc / jnp.maximum(cnt, 1.0)

    out = jax.lax.dot_general(agg, w_ref[...], (((1,), (0,)), ((), ())),
                              preferred_element_type=jnp.float32,
                              precision=_PREC_REF) + b_ref[...]
    if relu:
        out = jnp.maximum(out, 0.0)
    out_ref[...] = out

    if last:
        yb = y_ref[...]
        d2 = (out - yb) ** 2
        part = jnp.sum(jnp.sum(d2, axis=1, keepdims=True), axis=0,
                       keepdims=True)
        prev = jnp.where(i == 0, jnp.zeros((1, 1), jnp.float32),
                         loss_ref[...])
        total = prev + part
        loss_ref[...] = jnp.where(i == pl.num_programs(0) - 1,
                                  total / jnp.float32(N * 3), total)


def _layer_kernel(lo_ref, span_ref, hr_ref, ha_ref, brow_ref, bcol_ref,
                  w_ref, b_ref, y_ref, out_ref, loss_ref, keys_ref, *,
                  relu, last):
    i = pl.program_id(0)
    hr = hr_ref[...]
    sqr = jnp.sum(hr * hr, axis=1, keepdims=True)
    br = brow_ref[...]
    body = functools.partial(_phases, i, hr, sqr, br, ha_ref, bcol_ref,
                             keys_ref, w_ref, b_ref, y_ref, out_ref,
                             loss_ref, relu=relu, last=last)
    fits = span_ref[i] <= WCHUNK * C

    @pl.when(fits)
    def _windowed():
        body(lo=lo_ref[i], nchunk=WCHUNK)

    @pl.when(jnp.logical_not(fits))
    def _full():
        body(lo=jnp.int32(0), nchunk=NCHUNK)


def _layer(h, brow, bcol, lo_al, span_al, w, b, y, relu, last):
    kern = functools.partial(_layer_kernel, relu=relu, last=last)
    grid_spec = pltpu.PrefetchScalarGridSpec(
        num_scalar_prefetch=2,
        grid=(N // R,),
        in_specs=[
            pl.BlockSpec((R, D), lambda i, *_: (i, 0)),
            pl.BlockSpec((N, D), lambda i, *_: (0, 0)),
            pl.BlockSpec((R, 1), lambda i, *_: (i, 0)),
            pl.BlockSpec((1, N), lambda i, *_: (0, 0)),
            pl.BlockSpec((D, D), lambda i, *_: (0, 0)),
            pl.BlockSpec((1, D), lambda i, *_: (0, 0)),
            pl.BlockSpec((R, D), lambda i, *_: (i, 0)),
        ],
        out_specs=[
            pl.BlockSpec((R, D), lambda i, *_: (i, 0)),
            pl.BlockSpec((1, 1), lambda i, *_: (0, 0)),
        ],
        scratch_shapes=[pltpu.VMEM((R, N), jnp.int32)],
    )
    out_shape = [
        jax.ShapeDtypeStruct((N, D), jnp.float32),
        jax.ShapeDtypeStruct((1, 1), jnp.float32),
    ]
    return pl.pallas_call(kern, grid_spec=grid_spec, out_shape=out_shape)(
        lo_al, span_al, h, h, brow, bcol, w, b, y)


def _pad_w(w):
    return jnp.pad(w, ((0, D - w.shape[0]), (0, D - w.shape[1])))


def _pad_b(b):
    return jnp.pad(b, (0, D - b.shape[0])).reshape(1, D)


def kernel(x, batch, y, W1, b1, W2, b2, W3, b3):
    h = jnp.pad(x, ((0, 0), (0, D - x.shape[1])))
    yp = jnp.pad(y, ((0, 0), (0, D - y.shape[1])))
    brow = batch.reshape(N, 1)
    bcol = batch.reshape(1, N)
    r0 = jnp.arange(0, N, R)
    b0 = batch[r0]
    b1_ = batch[r0 + R - 1]
    lo = jnp.searchsorted(batch, b0, side="left").astype(jnp.int32)
    hi = jnp.searchsorted(batch, b1_, side="right").astype(jnp.int32)
    lo_al = (lo // ALIGN) * ALIGN
    lo_al = jnp.minimum(lo_al, N - WCHUNK * C)
    span_al = hi - lo_al
    h1, _ = _layer(h, brow, bcol, lo_al, span_al, _pad_w(W1), _pad_b(b1), yp,
                   True, False)
    h2, _ = _layer(h1, brow, bcol, lo_al, span_al, _pad_w(W2), _pad_b(b2), yp,
                   True, False)
    h3, loss = _layer(h2, brow, bcol, lo_al, span_al, _pad_w(W3), _pad_b(b3),
                      yp, False, True)
    return h3[:, :3], loss[0, 0]


# speedup vs baseline: 26.9103x; 1.0508x over previous
import functools

import jax
import jax.numpy as jnp
import numpy as np
from jax.experimental import pallas as pl
from jax.experimental.pallas import tpu as pltpu

N = 8192
K = 32
D = 128
R = 256
C = 1024
NCHUNK = N // C
WCHUNK = 3
ALIGN = 512
IMAX = np.int32(0x7FFFFFFF)
_PREC = jax.lax.Precision.HIGHEST
_PREC_REF = jax.lax.Precision.DEFAULT


def _phases(i, hr, sqr, br, ha_ref, bcol_ref, keys_ref, w_ref, b_ref, y_ref,
            out_ref, loss_ref, lo, nchunk, relu, last):
    ones = jnp.ones((1, D), jnp.float32)

    for ci in range(nchunk):
        off = pl.multiple_of(lo + ci * C, ALIGN)
        ha_c = ha_ref[pl.ds(off, C), :]
        g = jax.lax.dot_general(hr, ha_c, (((1,), (1,)), ((), ())),
                                preferred_element_type=jnp.float32,
                                precision=_PREC_REF)
        sqc = jax.lax.dot_general(ones, ha_c * ha_c, (((1,), (1,)), ((), ())),
                                  preferred_element_type=jnp.float32,
                                  precision=_PREC)
        dist = sqr + sqc - 2.0 * g
        u = jax.lax.bitcast_convert_type(dist, jnp.int32)
        key = u ^ ((u >> 31) & IMAX)
        bc = bcol_ref[0:1, pl.ds(off, C)]
        col_ids = off + jax.lax.broadcasted_iota(jnp.int32, (R, C), 1)
        row_ids = i * R + jax.lax.broadcasted_iota(jnp.int32, (R, C), 0)
        valid = (br == bc) & (col_ids != row_ids)
        keys_ref[:, ci * C:(ci + 1) * C] = jnp.where(valid, key, IMAX)

    def count_le(t):
        c = jnp.zeros((R, 1), jnp.int32)
        for ci in range(nchunk):
            kc = keys_ref[:, ci * C:(ci + 1) * C]
            c = c + jnp.sum((kc <= t).astype(jnp.int32), axis=1,
                            keepdims=True)
        return c

    mc = keys_ref[:, 0:C]
    for ci in range(1, nchunk):
        mc = jnp.minimum(mc, keys_ref[:, ci * C:(ci + 1) * C])
    w = C
    while w > 32:
        w //= 2
        mc = jnp.minimum(mc[:, :w], mc[:, w:2 * w])
    ub = jnp.max(mc, axis=1, keepdims=True)
    lb = jnp.min(mc, axis=1, keepdims=True)

    def bi_round(lo_, hi_, v_, res):
        d = hi_ - lo_
        mid = lo_ + ((d >> 1) & IMAX)
        c = count_le(mid)
        hit = jnp.logical_and(c == K, res == 0)
        v_ = jnp.where(hit, mid, v_)
        res = jnp.where(hit, jnp.int32(1), res)
        lt = c < K
        lo_ = jnp.where(lt, mid, lo_)
        hi_ = jnp.where(lt, hi_, mid)
        return lo_, hi_, v_, res

    def bi_cond(carry):
        it, _, _, _, res = carry
        return jnp.logical_and(it < 17, jnp.sum(res) < R)

    def bi_body(carry):
        it, lo_, hi_, v_, res = carry
        lo_, hi_, v_, res = bi_round(lo_, hi_, v_, res)
        lo_, hi_, v_, res = bi_round(lo_, hi_, v_, res)
        return it + 1, lo_, hi_, v_, res

    zero = jnp.zeros((R, 1), jnp.int32)
    _, _, hi_f, v, res_f = jax.lax.while_loop(
        bi_cond, bi_body, (jnp.int32(0), lb - 1, ub, zero, zero))
    v = jnp.where(res_f == 1, v, hi_f)

    acc = jnp.zeros((R, D), jnp.float32)
    cnt = jnp.zeros((R, 1), jnp.float32)
    for ci in range(nchunk):
        kc = keys_ref[:, ci * C:(ci + 1) * C]
        mc = ((kc <= v) & (kc != IMAX)).astype(jnp.float32)
        cnt = cnt + jnp.sum(mc, axis=1, keepdims=True)
        ha_c = ha_ref[pl.ds(pl.multiple_of(lo + ci * C, ALIGN), C), :]
        acc = acc + jax.lax.dot_general(mc, ha_c, (((1,), (0,)), ((), ())),
                                        preferred_element_type=jnp.float32,
                                        precision=_PREC)
    agg = acc / jnp.maximum(cnt, 1.0)

    out = jax.lax.dot_general(agg, w_ref[...], (((1,), (0,)), ((), ())),
                              preferred_element_type=jnp.float32,
                              precision=_PREC_REF) + b_ref[...]
    if relu:
        out = jnp.maximum(out, 0.0)
    out_ref[...] = out

    if last:
        yb = y_ref[...]
        d2 = (out - yb) ** 2
        part = jnp.sum(jnp.sum(d2, axis=1, keepdims=True), axis=0,
                       keepdims=True)
        prev = jnp.where(i == 0, jnp.zeros((1, 1), jnp.float32),
                         loss_ref[...])
        total = prev + part
        loss_ref[...] = jnp.where(i == pl.num_programs(0) - 1,
                                  total / jnp.float32(N * 3), total)


def _layer_kernel(lo_ref, span_ref, hr_ref, ha_ref, brow_ref, bcol_ref,
                  w_ref, b_ref, y_ref, out_ref, loss_ref, keys_ref, *,
                  relu, last):
    i = pl.program_id(0)
    hr = hr_ref[...]
    sqr = jnp.sum(hr * hr, axis=1, keepdims=True)
    br = brow_ref[...]
    body = functools.partial(_phases, i, hr, sqr, br, ha_ref, bcol_ref,
                             keys_ref, w_ref, b_ref, y_ref, out_ref,
                             loss_ref, relu=relu, last=last)
    fits = span_ref[i] <= WCHUNK * C

    @pl.when(fits)
    def _windowed():
        body(lo=lo_ref[i], nchunk=WCHUNK)

    @pl.when(jnp.logical_not(fits))
    def _full():
        body(lo=jnp.int32(0), nchunk=NCHUNK)


def _layer(h, brow, bcol, lo_al, span_al, w, b, y, relu, last):
    kern = functools.partial(_layer_kernel, relu=relu, last=last)
    grid_spec = pltpu.PrefetchScalarGridSpec(
        num_scalar_prefetch=2,
        grid=(N // R,),
        in_specs=[
            pl.BlockSpec((R, D), lambda i, *_: (i, 0)),
            pl.BlockSpec((N, D), lambda i, *_: (0, 0)),
            pl.BlockSpec((R, 1), lambda i, *_: (i, 0)),
            pl.BlockSpec((1, N), lambda i, *_: (0, 0)),
            pl.BlockSpec((D, D), lambda i, *_: (0, 0)),
            pl.BlockSpec((1, D), lambda i, *_: (0, 0)),
            pl.BlockSpec((R, D), lambda i, *_: (i, 0)),
        ],
        out_specs=[
            pl.BlockSpec((R, D), lambda i, *_: (i, 0)),
            pl.BlockSpec((1, 1), lambda i, *_: (0, 0)),
        ],
        scratch_shapes=[pltpu.VMEM((R, N), jnp.int32)],
    )
    out_shape = [
        jax.ShapeDtypeStruct((N, D), jnp.float32),
        jax.ShapeDtypeStruct((1, 1), jnp.float32),
    ]
    return pl.pallas_call(kern, grid_spec=grid_spec, out_shape=out_shape)(
        lo_al, span_al, h, h, brow, bcol, w, b, y)


def _pad_w(w):
    return jnp.pad(w, ((0, D - w.shape[0]), (0, D - w.shape[1])))


def _pad_b(b):
    return jnp.pad(b, (0, D - b.shape[0])).reshape(1, D)


def kernel(x, batch, y, W1, b1, W2, b2, W3, b3):
    h = jnp.pad(x, ((0, 0), (0, D - x.shape[1])))
    yp = jnp.pad(y, ((0, 0), (0, D - y.shape[1])))
    brow = batch.reshape(N, 1)
    bcol = batch.reshape(1, N)
    r0 = jnp.arange(0, N, R)
    b0 = batch[r0]
    b1_ = batch[r0 + R - 1]
    lo = jnp.searchsorted(batch, b0, side="left").astype(jnp.int32)
    hi = jnp.searchsorted(batch, b1_, side="right").astype(jnp.int32)
    lo_al = (lo // ALIGN) * ALIGN
    lo_al = jnp.minimum(lo_al, N - WCHUNK * C)
    span_al = hi - lo_al
    h1, _ = _layer(h, brow, bcol, lo_al, span_al, _pad_w(W1), _pad_b(b1), yp,
                   True, False)
    h2, _ = _layer(h1, brow, bcol, lo_al, span_al, _pad_w(W2), _pad_b(b2), yp,
                   True, False)
    h3, loss = _layer(h2, brow, bcol, lo_al, span_al, _pad_w(W3), _pad_b(b3),
                      yp, False, True)
    return h3[:, :3], loss[0, 0]
